# baseline (speedup 1.0000x reference)
import jax
import jax.numpy as jnp
from jax import lax
from jax.experimental import pallas as pl
from jax.experimental.pallas import tpu as pltpu
from jax.experimental.pallas import tpu_sc as plsc

N_NODES = 50000
D = 64
E = 800000
CH = 512
GRP = 1024
NW = 32
E_PAD = 819200
E_PAD2 = E_PAD // 2
W_GRPS = E_PAD // NW // GRP
T_GRPS = E_PAD // 16 // GRP
N_HALF = 25000
HALF_PAD = 25088
ROWS_PER_TILE = HALF_PAD // 16
NP_OUT = 2 * HALF_PAD
CNT_ROWS = 2 * HALF_PAD
IDX_ROWS = E_PAD // NW // 128
HB = 128


def _zero16():
    return jnp.zeros((16,), jnp.float32)


def _pq_body(f_ref, p_ref, w1a, w1b, w1c, b1, P_ref, Q_ref):
    f = f_ref[...]
    pc = jnp.dot(p_ref[...], w1c[...], preferred_element_type=jnp.float32)
    P_ref[...] = (jnp.dot(f, w1a[...], preferred_element_type=jnp.float32)
                  + pc + b1[...]).astype(jnp.bfloat16)
    Q_ref[...] = (jnp.dot(f, w1b[...], preferred_element_type=jnp.float32)
                  - pc).astype(jnp.bfloat16)


SLOW_C = 1
G_SLOW = 24
G_FAST = 26
N_GRPS = E_PAD // GRP


def _gather_body(p_hbm, q_hbm, src_hbm, dst_hbm, e_hbm,
                 idxs0, idxd0, idxs1, idxd1, buf1, buf2, sem0, sem1, semg):
    c = lax.axis_index("c")
    s = lax.axis_index("s")
    slow = c == SLOW_C
    gbase = jnp.where(slow, s * G_SLOW, 16 * G_SLOW + s * G_FAST)
    niter = jnp.where(slow, G_SLOW // 2, G_FAST // 2)

    def idx_load(g, bs, bd, sem):
        r0 = pl.multiple_of(g * 8, 8)
        cps = [pltpu.async_copy(src_hbm.at[pl.ds(r0, 8)], bs, sem),
               pltpu.async_copy(dst_hbm.at[pl.ds(r0, 8)], bd, sem)]
        return cps

    def idx_wait(bs, bd, sem):
        pltpu.make_async_copy(src_hbm.at[pl.ds(0, 8)], bs, sem).wait()
        pltpu.make_async_copy(dst_hbm.at[pl.ds(0, 8)], bd, sem).wait()

    def process(g, bs, bd):
        for hf in range(2):
            e0 = pl.multiple_of(g * GRP + hf * CH, 512)
            cps = []
            for j in range(4):
                cps.append(pltpu.async_copy(
                    p_hbm.at[bs.at[hf * 4 + j]],
                    buf1.at[pl.ds(j * 128, 128)], semg))
                cps.append(pltpu.async_copy(
                    q_hbm.at[bd.at[hf * 4 + j]],
                    buf2.at[pl.ds(j * 128, 128)], semg))
            for cp in cps:
                cp.wait()

            def rowop(j, _):
                for k in range(2):
                    sl = pl.ds(k * 32, 32)
                    buf1[j, sl] = jnp.maximum(buf1[j, sl] + buf2[j, sl],
                                              jnp.bfloat16(0.0))
                return 0
            lax.fori_loop(0, CH, rowop, 0)
            pltpu.sync_copy(buf1, e_hbm.at[pl.ds(e0, CH)])

    for cp in idx_load(gbase, idxs0, idxd0, sem0):
        cp.wait()

    def body(t, _):
        g0 = gbase + 2 * t
        idx_load(g0 + 1, idxs1, idxd1, sem1)
        process(g0, idxs0, idxd0)
        idx_wait(idxs1, idxd1, sem1)
        idx_load(g0 + 2, idxs0, idxd0, sem0)
        process(g0 + 1, idxs1, idxd1)
        idx_wait(idxs0, idxd0, sem0)
        return 0

    lax.fori_loop(0, niter, body, 0)


def _edge_mlp_body(e_ref, w2, b2, h_ref):
    e = e_ref[...].astype(jnp.float32)
    h = jnp.dot(e, w2[...], preferred_element_type=jnp.float32) + b2[...]
    h_ref[...] = jnp.maximum(h, 0.0)


def _segsum_body(h_hbm, dst_hbm, ones_hbm, zeros_hbm, sums_hbm, cnt_hbm,
                 hbuf, dstb, locb, onesv, shs, shc, sem):
    c = lax.axis_index("c")
    s = lax.axis_index("s")
    nbase = c * N_HALF

    pltpu.sync_copy(ones_hbm, onesv)

    def z_h(r, _):
        for k in range(4):
            hbuf[r, pl.ds(k * 16, 16)] = _zero16()
        return 0
    lax.fori_loop(0, HB, z_h, 0)

    rb = pl.multiple_of(s * ROWS_PER_TILE, 32)
    for t in range(12):
        pltpu.sync_copy(hbuf, shs.at[pl.ds(rb + t * HB, HB)])
        pltpu.sync_copy(zeros_hbm, shc.at[pl.ds(rb + t * HB, HB)])
    pltpu.sync_copy(hbuf.at[pl.ds(0, 32)], shs.at[pl.ds(rb + 1536, 32)])
    pltpu.sync_copy(zeros_hbm.at[pl.ds(0, 32)], shc.at[pl.ds(rb + 1536, 32)])
    plsc.subcore_barrier()

    tbase = s * (E_PAD // 16)

    def group(i, _):
        e0 = pl.multiple_of(tbase + i * GRP, 512)
        r0 = pl.multiple_of(tbase // 128 + i * 8, 8)
        pltpu.sync_copy(dst_hbm.at[pl.ds(r0, 8)], dstb)
        for j in range(8):
            for k in range(8):
                v = dstb[j, pl.ds(k * 16, 16)] - nbase
                inr = (v >= 0) & (v < N_HALF)
                locb[j, pl.ds(k * 16, 16)] = jnp.where(inr, v, N_HALF)
        for j in range(8):
            pltpu.sync_copy(h_hbm.at[pl.ds(e0 + j * HB, HB)], hbuf)
            pltpu.sync_copy(hbuf, shs.at[locb.at[j]], add=True)
            pltpu.sync_copy(onesv, shc.at[locb.at[j]], add=True)
        return 0

    lax.fori_loop(0, T_GRPS, group, 0)
    plsc.subcore_barrier()

    ob = pl.multiple_of(c * HALF_PAD + rb, 32)
    for t in range(12):
        pltpu.sync_copy(shs.at[pl.ds(rb + t * HB, HB)],
                        sums_hbm.at[pl.ds(ob + t * HB, HB)])
    pltpu.sync_copy(shs.at[pl.ds(rb + 1536, 32)],
                    sums_hbm.at[pl.ds(ob + 1536, 32)])

    @pl.when(s == 0)
    def _():
        cb = pl.multiple_of(c * HALF_PAD, 32)
        pltpu.sync_copy(shc, cnt_hbm.at[pl.ds(cb, HALF_PAD)])


def _out_body(sums_ref, cnt_ref, f_ref, w3, b3, w4, b4, o_ref):
    sc = sums_ref[...]
    cnt = cnt_ref[...]
    cl = jnp.maximum(cnt[:, 0:1], 1.0)
    cr = jnp.maximum(cnt[:, 8:9], 1.0)
    div = jnp.concatenate([jnp.broadcast_to(cl, (cl.shape[0], D)),
                           jnp.broadcast_to(cr, (cr.shape[0], D))], axis=-1)
    agg = sc / div
    u = jnp.maximum(jnp.dot(agg, w3[...], preferred_element_type=jnp.float32) + b3[...], 0.0)
    u = jnp.maximum(jnp.dot(u, w4[...], preferred_element_type=jnp.float32) + b4[...], 0.0)
    o_ref[...] = u + f_ref[...]


def _bdiag(w):
    z = jnp.zeros_like(w)
    return jnp.concatenate([jnp.concatenate([w, z], axis=1),
                            jnp.concatenate([z, w], axis=1)], axis=0)


def kernel(features, points, l0_edges, W1, b1, W2, b2, W3, b3, W4, b4):
    src = l0_edges[:, 0].astype(jnp.int32)
    dst = l0_edges[:, 1].astype(jnp.int32)
    src_p = jnp.concatenate([src, jnp.zeros((E_PAD + GRP - E,), jnp.int32)])
    dst_p = jnp.concatenate([dst, jnp.full((E_PAD + GRP - E,), N_NODES, jnp.int32)])
    src2d = src_p.reshape((E_PAD + GRP) // 128, 128)
    dst2d = dst_p.reshape((E_PAD + GRP) // 128, 128)

    feat128 = features.reshape(N_NODES // 2, 2 * D)
    pts64 = jnp.pad(points, ((0, 0), (0, D - points.shape[1])))
    pts128 = pts64.reshape(N_NODES // 2, 2 * D)
    w1a = _bdiag(W1[:D])
    w1b = _bdiag(W1[D:2 * D])
    w1c = _bdiag(jnp.pad(W1[2 * D:], ((0, D - (W1.shape[0] - 2 * D)), (0, 0))))
    w2d = _bdiag(W2)
    w3d = _bdiag(W3)
    w4d = _bdiag(W4)
    b1p = jnp.concatenate([b1, b1]).reshape(1, 2 * D)
    b2p = jnp.concatenate([b2, b2]).reshape(1, 2 * D)
    b3p = jnp.concatenate([b3, b3]).reshape(1, 2 * D)
    b4p = jnp.concatenate([b4, b4]).reshape(1, 2 * D)

    NB = 1000
    const = pl.BlockSpec((2 * D, 2 * D), lambda i: (0, 0))
    bias = pl.BlockSpec((1, 2 * D), lambda i: (0, 0))
    row = pl.BlockSpec((NB, 2 * D), lambda i: (i, 0))

    feat128p = jnp.pad(feat128, ((0, HALF_PAD - N_NODES // 2), (0, 0)))
    pts128p = jnp.pad(pts128, ((0, HALF_PAD - N_NODES // 2), (0, 0)))
    rowP = pl.BlockSpec((784, 2 * D), lambda i: (i, 0))
    P2, Q2 = pl.pallas_call(
        _pq_body,
        grid=(HALF_PAD // 784,),
        in_specs=[rowP, rowP, const, const, const, bias],
        out_specs=[rowP, rowP],
        out_shape=[jax.ShapeDtypeStruct((HALF_PAD, 2 * D), jnp.bfloat16)] * 2,
        compiler_params=pltpu.CompilerParams(
            dimension_semantics=("arbitrary",)),
    )(feat128p, pts128p, w1a, w1b, w1c, b1p)
    P = P2.reshape(2 * HALF_PAD, D)
    Q = Q2.reshape(2 * HALF_PAD, D)

    mesh = plsc.VectorSubcoreMesh(core_axis_name="c", subcore_axis_name="s")
    e_arr = pl.kernel(
        _gather_body,
        out_type=jax.ShapeDtypeStruct((E_PAD, D), jnp.bfloat16),
        mesh=mesh,
        scratch_types=[
            pltpu.VMEM((8, 128), jnp.int32),
            pltpu.VMEM((8, 128), jnp.int32),
            pltpu.VMEM((8, 128), jnp.int32),
            pltpu.VMEM((8, 128), jnp.int32),
            pltpu.VMEM((CH, D), jnp.bfloat16),
            pltpu.VMEM((CH, D), jnp.bfloat16),
            pltpu.SemaphoreType.DMA,
            pltpu.SemaphoreType.DMA,
            pltpu.SemaphoreType.DMA,
        ],
        compiler_params=pltpu.CompilerParams(use_tc_tiling_on_sc=False),
    )(P, Q, src2d, dst2d)

    EB = 1024
    e128 = e_arr.reshape(E_PAD2, 2 * D)
    h128 = pl.pallas_call(
        _edge_mlp_body,
        grid=(E_PAD2 // EB,),
        in_specs=[pl.BlockSpec((EB, 2 * D), lambda i: (i, 0)),
                  const, bias],
        out_specs=pl.BlockSpec((EB, 2 * D), lambda i: (i, 0)),
        out_shape=jax.ShapeDtypeStruct((E_PAD2, 2 * D), jnp.float32),
        compiler_params=pltpu.CompilerParams(
            dimension_semantics=("arbitrary",)),
    )(e128, w2d, b2p)
    h64 = h128.reshape(E_PAD, D)

    ones8 = jnp.ones((HB, 8), jnp.float32)
    zeros8 = jnp.zeros((HB, 8), jnp.float32)
    sums_raw, cnt_raw = pl.kernel(
        _segsum_body,
        out_type=[jax.ShapeDtypeStruct((NP_OUT, D), jnp.float32),
                  jax.ShapeDtypeStruct((CNT_ROWS, 8), jnp.float32)],
        mesh=mesh,
        scratch_types=[
            pltpu.VMEM((HB, D), jnp.float32),
            pltpu.VMEM((8, 128), jnp.int32),
            pltpu.VMEM((8, 128), jnp.int32),
            pltpu.VMEM((HB, 8), jnp.float32),
            pltpu.VMEM_SHARED((HALF_PAD, D), jnp.float32),
            pltpu.VMEM_SHARED((HALF_PAD, 8), jnp.float32),
            pltpu.SemaphoreType.DMA,
        ],
        compiler_params=pltpu.CompilerParams(use_tc_tiling_on_sc=False,
                                             needs_layout_passes=False),
    )(h64, dst2d, ones8, zeros8)

    SR = sums_raw.reshape(HALF_PAD, 2 * D)
    sums128 = jnp.concatenate([SR[:N_HALF // 2],
                               SR[HALF_PAD // 2:HALF_PAD // 2 + N_HALF // 2]])
    CR = cnt_raw.reshape(HALF_PAD, 16)
    cnt16 = jnp.concatenate([CR[:N_HALF // 2],
                             CR[HALF_PAD // 2:HALF_PAD // 2 + N_HALF // 2]])

    out128 = pl.pallas_call(
        _out_body,
        grid=(N_NODES // 2 // NB,),
        in_specs=[row, pl.BlockSpec((NB, 16), lambda i: (i, 0)), row,
                  const, bias, const, bias],
        out_specs=row,
        out_shape=jax.ShapeDtypeStruct((N_NODES // 2, 2 * D), jnp.float32),
        compiler_params=pltpu.CompilerParams(
            dimension_semantics=("arbitrary",)),
    )(sums128, cnt16, feat128, w3d, b3p, w4d, b4p)
    return out128.reshape(N_NODES, D)


# speedup vs baseline: 1.0354x; 1.0354x over previous
import jax
import jax.numpy as jnp
from jax import lax
from jax.experimental import pallas as pl
from jax.experimental.pallas import tpu as pltpu
from jax.experimental.pallas import tpu_sc as plsc

N_NODES = 50000
D = 64
E = 800000
CH = 512
GRP = 1024
NW = 32
E_PAD = 819200
E_PAD2 = E_PAD // 2
W_GRPS = E_PAD // NW // GRP
T_GRPS = E_PAD // 16 // GRP
N_HALF = 25000
HALF_PAD = 25088
ROWS_PER_TILE = HALF_PAD // 16
NP_OUT = 2 * HALF_PAD
CNT_ROWS = 2 * HALF_PAD
IDX_ROWS = E_PAD // NW // 128
HB = 128


def _zero16():
    return jnp.zeros((16,), jnp.float32)


def _pq_body(f_ref, p_ref, w1a, w1b, w1c, b1, P_ref, Q_ref):
    f = f_ref[...]
    pc = jnp.dot(p_ref[...], w1c[...], preferred_element_type=jnp.float32)
    P_ref[...] = jnp.dot(f, w1a[...], preferred_element_type=jnp.float32) + pc + b1[...]
    Q_ref[...] = jnp.dot(f, w1b[...], preferred_element_type=jnp.float32) - pc


def _gather_body(p_hbm, q_hbm, src_hbm, dst_hbm, e_hbm,
                 idxs, idxd, buf1, buf2, sem):
    c = lax.axis_index("c")
    s = lax.axis_index("s")
    w = s * 2 + c
    wbase = w * (E_PAD // NW)
    rbase = pl.multiple_of(w * IDX_ROWS, 8)
    pltpu.sync_copy(src_hbm.at[pl.ds(rbase, IDX_ROWS)], idxs)
    pltpu.sync_copy(dst_hbm.at[pl.ds(rbase, IDX_ROWS)], idxd)

    def half(hf, _):
        e0 = pl.multiple_of(wbase + hf * CH, 512)
        r0 = hf * 4
        cps = []
        for j in range(4):
            cps.append(pltpu.async_copy(
                p_hbm.at[idxs.at[r0 + j]], buf1.at[pl.ds(j * 128, 128)], sem))
            cps.append(pltpu.async_copy(
                q_hbm.at[idxd.at[r0 + j]], buf2.at[pl.ds(j * 128, 128)], sem))
        for cp in cps:
            cp.wait()

        def rowop(j, _):
            for k in range(4):
                sl = pl.ds(k * 16, 16)
                buf1[j, sl] = jnp.maximum(buf1[j, sl] + buf2[j, sl], 0.0)
            return 0
        lax.fori_loop(0, CH, rowop, 0)
        pltpu.sync_copy(buf1, e_hbm.at[pl.ds(e0, CH)])
        return 0

    lax.fori_loop(0, 2 * W_GRPS, half, 0)


def _edge_mlp_body(e_ref, w2, b2, h_ref):
    h = jnp.dot(e_ref[...], w2[...], preferred_element_type=jnp.float32) + b2[...]
    h_ref[...] = jnp.maximum(h, 0.0)


def _segsum_body(h_hbm, dst_hbm, ones_hbm, zeros_hbm, sums_hbm, cnt_hbm,
                 hbuf, dstb, locb, onesv, shs, shc, sem):
    c = lax.axis_index("c")
    s = lax.axis_index("s")
    nbase = c * N_HALF

    pltpu.sync_copy(ones_hbm, onesv)

    def z_h(r, _):
        for k in range(4):
            hbuf[r, pl.ds(k * 16, 16)] = _zero16()
        return 0
    lax.fori_loop(0, HB, z_h, 0)

    rb = pl.multiple_of(s * ROWS_PER_TILE, 32)
    for t in range(12):
        pltpu.sync_copy(hbuf, shs.at[pl.ds(rb + t * HB, HB)])
        pltpu.sync_copy(zeros_hbm, shc.at[pl.ds(rb + t * HB, HB)])
    pltpu.sync_copy(hbuf.at[pl.ds(0, 32)], shs.at[pl.ds(rb + 1536, 32)])
    pltpu.sync_copy(zeros_hbm.at[pl.ds(0, 32)], shc.at[pl.ds(rb + 1536, 32)])
    plsc.subcore_barrier()

    tbase = s * (E_PAD // 16)

    def group(i, _):
        e0 = pl.multiple_of(tbase + i * GRP, 512)
        r0 = pl.multiple_of(tbase // 128 + i * 8, 8)
        pltpu.sync_copy(dst_hbm.at[pl.ds(r0, 8)], dstb)
        for j in range(8):
            for k in range(8):
                v = dstb[j, pl.ds(k * 16, 16)] - nbase
                inr = (v >= 0) & (v < N_HALF)
                locb[j, pl.ds(k * 16, 16)] = jnp.where(inr, v, N_HALF)
        for j in range(8):
            pltpu.sync_copy(h_hbm.at[pl.ds(e0 + j * HB, HB)], hbuf)
            pltpu.sync_copy(hbuf, shs.at[locb.at[j]], add=True)
            pltpu.sync_copy(onesv, shc.at[locb.at[j]], add=True)
        return 0

    lax.fori_loop(0, T_GRPS, group, 0)
    plsc.subcore_barrier()

    ob = pl.multiple_of(c * HALF_PAD + rb, 32)
    for t in range(12):
        pltpu.sync_copy(shs.at[pl.ds(rb + t * HB, HB)],
                        sums_hbm.at[pl.ds(ob + t * HB, HB)])
    pltpu.sync_copy(shs.at[pl.ds(rb + 1536, 32)],
                    sums_hbm.at[pl.ds(ob + 1536, 32)])

    @pl.when(s == 0)
    def _():
        cb = pl.multiple_of(c * HALF_PAD, 32)
        pltpu.sync_copy(shc, cnt_hbm.at[pl.ds(cb, HALF_PAD)])


def _out_body(sums_ref, cnt_ref, f_ref, w3, b3, w4, b4, o_ref):
    sc = sums_ref[...]
    cnt = cnt_ref[...]
    cl = jnp.maximum(cnt[:, 0:1], 1.0)
    cr = jnp.maximum(cnt[:, 8:9], 1.0)
    div = jnp.concatenate([jnp.broadcast_to(cl, (cl.shape[0], D)),
                           jnp.broadcast_to(cr, (cr.shape[0], D))], axis=-1)
    agg = sc / div
    u = jnp.maximum(jnp.dot(agg, w3[...], preferred_element_type=jnp.float32) + b3[...], 0.0)
    u = jnp.maximum(jnp.dot(u, w4[...], preferred_element_type=jnp.float32) + b4[...], 0.0)
    o_ref[...] = u + f_ref[...]


def _bdiag(w):
    z = jnp.zeros_like(w)
    return jnp.concatenate([jnp.concatenate([w, z], axis=1),
                            jnp.concatenate([z, w], axis=1)], axis=0)


def kernel(features, points, l0_edges, W1, b1, W2, b2, W3, b3, W4, b4):
    src = l0_edges[:, 0].astype(jnp.int32)
    dst = l0_edges[:, 1].astype(jnp.int32)
    src_p = jnp.concatenate([src, jnp.zeros((E_PAD + GRP - E,), jnp.int32)])
    dst_p = jnp.concatenate([dst, jnp.full((E_PAD + GRP - E,), N_NODES, jnp.int32)])
    src2d = src_p.reshape((E_PAD + GRP) // 128, 128)
    dst2d = dst_p.reshape((E_PAD + GRP) // 128, 128)

    feat128 = features.reshape(N_NODES // 2, 2 * D)
    pts64 = jnp.pad(points, ((0, 0), (0, D - points.shape[1])))
    pts128 = pts64.reshape(N_NODES // 2, 2 * D)
    w1a = _bdiag(W1[:D])
    w1b = _bdiag(W1[D:2 * D])
    w1c = _bdiag(jnp.pad(W1[2 * D:], ((0, D - (W1.shape[0] - 2 * D)), (0, 0))))
    w2d = _bdiag(W2)
    w3d = _bdiag(W3)
    w4d = _bdiag(W4)
    b1p = jnp.concatenate([b1, b1]).reshape(1, 2 * D)
    b2p = jnp.concatenate([b2, b2]).reshape(1, 2 * D)
    b3p = jnp.concatenate([b3, b3]).reshape(1, 2 * D)
    b4p = jnp.concatenate([b4, b4]).reshape(1, 2 * D)

    NB = 1000
    const = pl.BlockSpec((2 * D, 2 * D), lambda i: (0, 0))
    bias = pl.BlockSpec((1, 2 * D), lambda i: (0, 0))
    row = pl.BlockSpec((NB, 2 * D), lambda i: (i, 0))

    P2, Q2 = pl.pallas_call(
        _pq_body,
        grid=(N_NODES // 2 // NB,),
        in_specs=[row, row, const, const, const, bias],
        out_specs=[row, row],
        out_shape=[jax.ShapeDtypeStruct((HALF_PAD, 2 * D), jnp.float32)] * 2,
        compiler_params=pltpu.CompilerParams(
            dimension_semantics=("arbitrary",)),
    )(feat128, pts128, w1a, w1b, w1c, b1p)
    P = P2.reshape(2 * HALF_PAD, D)
    Q = Q2.reshape(2 * HALF_PAD, D)

    mesh = plsc.VectorSubcoreMesh(core_axis_name="c", subcore_axis_name="s")
    e_arr = pl.kernel(
        _gather_body,
        out_type=jax.ShapeDtypeStruct((E_PAD, D), jnp.float32),
        mesh=mesh,
        scratch_types=[
            pltpu.VMEM((IDX_ROWS, 128), jnp.int32),
            pltpu.VMEM((IDX_ROWS, 128), jnp.int32),
            pltpu.VMEM((CH, D), jnp.float32),
            pltpu.VMEM((CH, D), jnp.float32),
            pltpu.SemaphoreType.DMA,
        ],
        compiler_params=pltpu.CompilerParams(use_tc_tiling_on_sc=False),
    )(P, Q, src2d, dst2d)

    EB = 1024
    e128 = e_arr.reshape(E_PAD2, 2 * D)
    h128 = pl.pallas_call(
        _edge_mlp_body,
        grid=(E_PAD2 // EB,),
        in_specs=[pl.BlockSpec((EB, 2 * D), lambda i: (i, 0)),
                  const, bias],
        out_specs=pl.BlockSpec((EB, 2 * D), lambda i: (i, 0)),
        out_shape=jax.ShapeDtypeStruct((E_PAD2, 2 * D), jnp.float32),
        compiler_params=pltpu.CompilerParams(
            dimension_semantics=("arbitrary",)),
    )(e128, w2d, b2p)
    h64 = h128.reshape(E_PAD, D)

    ones8 = jnp.ones((HB, 8), jnp.float32)
    zeros8 = jnp.zeros((HB, 8), jnp.float32)
    sums_raw, cnt_raw = pl.kernel(
        _segsum_body,
        out_type=[jax.ShapeDtypeStruct((NP_OUT, D), jnp.float32),
                  jax.ShapeDtypeStruct((CNT_ROWS, 8), jnp.float32)],
        mesh=mesh,
        scratch_types=[
            pltpu.VMEM((HB, D), jnp.float32),
            pltpu.VMEM((8, 128), jnp.int32),
            pltpu.VMEM((8, 128), jnp.int32),
            pltpu.VMEM((HB, 8), jnp.float32),
            pltpu.VMEM_SHARED((HALF_PAD, D), jnp.float32),
            pltpu.VMEM_SHARED((HALF_PAD, 8), jnp.float32),
            pltpu.SemaphoreType.DMA,
        ],
        compiler_params=pltpu.CompilerParams(use_tc_tiling_on_sc=False,
                                             needs_layout_passes=False),
    )(h64, dst2d, ones8, zeros8)

    SR = sums_raw.reshape(HALF_PAD, 2 * D)
    sums128 = jnp.concatenate([SR[:N_HALF // 2],
                               SR[HALF_PAD // 2:HALF_PAD // 2 + N_HALF // 2]])
    CR = cnt_raw.reshape(HALF_PAD, 16)
    cnt16 = jnp.concatenate([CR[:N_HALF // 2],
                             CR[HALF_PAD // 2:HALF_PAD // 2 + N_HALF // 2]])

    out128 = pl.pallas_call(
        _out_body,
        grid=(N_NODES // 2 // NB,),
        in_specs=[row, pl.BlockSpec((NB, 16), lambda i: (i, 0)), row,
                  const, bias, const, bias],
        out_specs=row,
        out_shape=jax.ShapeDtypeStruct((N_NODES // 2, 2 * D), jnp.float32),
        compiler_params=pltpu.CompilerParams(
            dimension_semantics=("arbitrary",)),
    )(sums128, cnt16, feat128, w3d, b3p, w4d, b4p)
    return out128.reshape(N_NODES, D)
